# Initial kernel scaffold; baseline (speedup 1.0000x reference)
import jax
import jax.numpy as jnp
from jax.experimental import pallas as pl


def kernel(ppg, keys, prompt, conv_w, conv_b):
    raise NotImplementedError("write your pallas kernel here")


# speedup vs baseline: 1.9699x; 1.9699x over previous
import functools

import jax
import jax.numpy as jnp
from jax import lax
from jax.experimental import pallas as pl
from jax.experimental.pallas import tpu as pltpu
from jax.experimental.pallas import tpu_sc as plsc

B = 1024
P = 100000
D = 128
K = 8
EPS = 1e-8
PB = 512
NB = (P + PB - 1) // PB
P_PAD = NB * PB
NEG = float("-inf")
BIGID = 1e9


def _scores_body(ppg2_ref, keys_ref, bv_ref, bi_ref, z_ref, s_ref,
                 ppgn_s, bv_s, bi_s, z_s, s_s):
    i = pl.program_id(0)

    @pl.when(i == 0)
    def _init():
        q = ppg2_ref[...]
        qn = jnp.maximum(jnp.sqrt(jnp.sum(q * q, axis=1, keepdims=True)), EPS)
        ppgn_s[...] = q / qn
        bv_s[...] = jnp.full((B, K), NEG, jnp.float32)
        bi_s[...] = jnp.zeros((B, K), jnp.float32)
        z_s[...] = jnp.zeros((B, D), jnp.float32)
        s_s[...] = jnp.zeros((B, D), jnp.float32)

    kblk = keys_ref[...]
    kn = jnp.maximum(jnp.sqrt(jnp.sum(kblk * kblk, axis=1, keepdims=True)), EPS)
    kbn = kblk / kn
    cos = lax.dot_general(ppgn_s[...], kbn, (((1,), (1,)), ((), ())),
                          preferred_element_type=jnp.float32)

    gid = (i * PB + lax.broadcasted_iota(jnp.int32, (B, PB), 1)
           ).astype(jnp.float32)
    valid = gid < float(P)

    t = -cos
    e = jnp.where(valid, jnp.exp(t), 0.0)
    te = t * e
    z_s[...] += (e[:, 0:128] + e[:, 128:256]) + (e[:, 256:384] + e[:, 384:512])
    s_s[...] += (te[:, 0:128] + te[:, 128:256]) + (te[:, 256:384] + te[:, 384:512])

    cm = jnp.where(valid, cos, NEG)
    vals, ids = [], []
    for _ in range(K):
        m = jnp.max(cm, axis=1, keepdims=True)
        am = jnp.min(jnp.where(cm == m, gid, BIGID), axis=1, keepdims=True)
        cm = jnp.where(gid == am, NEG, cm)
        vals.append(m)
        ids.append(am)
    bvals = jnp.concatenate(vals, axis=1)
    bids = jnp.concatenate(ids, axis=1)

    catv = jnp.concatenate([bv_s[...], bvals], axis=1)
    cati = jnp.concatenate([bi_s[...], bids], axis=1)
    nv, ni = [], []
    for _ in range(K):
        m = jnp.max(catv, axis=1, keepdims=True)
        am = jnp.min(jnp.where(catv == m, cati, BIGID), axis=1, keepdims=True)
        catv = jnp.where((catv == m) & (cati == am), NEG, catv)
        nv.append(m)
        ni.append(am)
    bv_s[...] = jnp.concatenate(nv, axis=1)
    bi_s[...] = jnp.concatenate(ni, axis=1)

    @pl.when(i == NB - 1)
    def _fin():
        bv_ref[...] = bv_s[...]
        bi_ref[...] = bi_s[...]
        z_ref[...] = jnp.sum(z_s[...], axis=1, keepdims=True)
        s_ref[...] = jnp.sum(s_s[...], axis=1, keepdims=True)


def _topk_entropy(ppg2, keys_pad, interpret=False):
    return pl.pallas_call(
        _scores_body,
        grid=(NB,),
        in_specs=[
            pl.BlockSpec((B, D), lambda i: (0, 0)),
            pl.BlockSpec((PB, D), lambda i: (i, 0)),
        ],
        out_specs=[
            pl.BlockSpec((B, K), lambda i: (0, 0)),
            pl.BlockSpec((B, K), lambda i: (0, 0)),
            pl.BlockSpec((B, 1), lambda i: (0, 0)),
            pl.BlockSpec((B, 1), lambda i: (0, 0)),
        ],
        out_shape=[
            jax.ShapeDtypeStruct((B, K), jnp.float32),
            jax.ShapeDtypeStruct((B, K), jnp.float32),
            jax.ShapeDtypeStruct((B, 1), jnp.float32),
            jax.ShapeDtypeStruct((B, 1), jnp.float32),
        ],
        scratch_shapes=[
            pltpu.VMEM((B, D), jnp.float32),
            pltpu.VMEM((B, K), jnp.float32),
            pltpu.VMEM((B, K), jnp.float32),
            pltpu.VMEM((B, D), jnp.float32),
            pltpu.VMEM((B, D), jnp.float32),
        ],
        interpret=interpret,
    )(ppg2, keys_pad)


def _make_sc_gather():
    NC, NS = 2, 16
    NW = NC * NS
    n_rows = B * K
    bpw = n_rows // NW
    mesh = plsc.VectorSubcoreMesh(core_axis_name="c", subcore_axis_name="s")

    @functools.partial(
        pl.kernel, mesh=mesh,
        out_type=jax.ShapeDtypeStruct((n_rows, D), jnp.float32),
        scratch_types=[
            pltpu.VMEM((bpw,), jnp.int32),
            pltpu.VMEM((bpw, D), jnp.float32),
            pltpu.SemaphoreType.DMA,
        ],
    )
    def gather_k(table_hbm, idx_hbm, out_hbm, idx_v, rows_v, sem):
        wid = lax.axis_index("s") * NC + lax.axis_index("c")
        base = wid * bpw
        pltpu.sync_copy(idx_hbm.at[pl.ds(base, bpw)], idx_v)
        pltpu.async_copy(table_hbm.at[idx_v], rows_v, sem).wait()
        pltpu.sync_copy(rows_v, out_hbm.at[pl.ds(base, bpw)])

    return gather_k


def _combine_body(g_ref, ppg2_ref, w_ref, b_ref, bv_ref, z_ref, s_ref,
                  out_ref, sm_ref, ent_ref, acc_s):
    k = pl.program_id(0)
    w0 = w_ref[0]
    w1 = w_ref[1]
    w2 = w_ref[2]
    bb = b_ref[0]
    x = g_ref[...]
    zcol = jnp.zeros((B, 1), jnp.float32)
    xl = jnp.concatenate([zcol, x[:, :D - 1]], axis=1)
    xr = jnp.concatenate([x[:, 1:], zcol], axis=1)
    y = jnp.maximum(w0 * xl + w1 * x + w2 * xr + bb, 0.0)

    @pl.when(k == 0)
    def _init():
        acc_s[...] = y

    @pl.when(k > 0)
    def _acc():
        acc_s[...] += y

    @pl.when(k == K - 1)
    def _fin():
        out_ref[...] = ppg2_ref[...] + acc_s[...] * jnp.float32(1.0 / K)
        sm_ref[...] = jnp.mean(1.0 - bv_ref[...]).reshape(1, 1)
        z = z_ref[...]
        s = s_ref[...]
        ent = jnp.log(z) - s / z
        ent_ref[...] = jnp.mean(ent).reshape(1, 1)


def _combine(gathered, ppg2, conv_w, conv_b, bv, z, s, interpret=False):
    return pl.pallas_call(
        _combine_body,
        grid=(K,),
        in_specs=[
            pl.BlockSpec((B, D), lambda k: (k, 0)),
            pl.BlockSpec((B, D), lambda k: (0, 0)),
            pl.BlockSpec(memory_space=pltpu.SMEM),
            pl.BlockSpec(memory_space=pltpu.SMEM),
            pl.BlockSpec((B, K), lambda k: (0, 0)),
            pl.BlockSpec((B, 1), lambda k: (0, 0)),
            pl.BlockSpec((B, 1), lambda k: (0, 0)),
        ],
        out_specs=[
            pl.BlockSpec((B, D), lambda k: (0, 0)),
            pl.BlockSpec((1, 1), lambda k: (0, 0)),
            pl.BlockSpec((1, 1), lambda k: (0, 0)),
        ],
        out_shape=[
            jax.ShapeDtypeStruct((B, D), jnp.float32),
            jax.ShapeDtypeStruct((1, 1), jnp.float32),
            jax.ShapeDtypeStruct((1, 1), jnp.float32),
        ],
        scratch_shapes=[pltpu.VMEM((B, D), jnp.float32)],
        interpret=interpret,
    )(gathered, ppg2, conv_w, conv_b, bv, z, s)


def kernel(ppg, keys, prompt, conv_w, conv_b):
    ppg2 = ppg[:, 0, :]
    keys_pad = jnp.pad(keys, ((0, P_PAD - P), (0, 0)))

    bv, bi, z, s = _topk_entropy(ppg2, keys_pad)

    idx = bi.astype(jnp.int32)
    idx_flat = idx.T.reshape(B * K)
    gathered = _make_sc_gather()(prompt, idx_flat)

    out2, sm, ent = _combine(gathered, ppg2, conv_w, conv_b, bv, z, s)
    return (out2[:, None, :], sm[0, 0], ent[0, 0])


# speedup vs baseline: 4.0485x; 2.0551x over previous
import functools

import jax
import jax.numpy as jnp
from jax import lax
from jax.experimental import pallas as pl
from jax.experimental.pallas import tpu as pltpu
from jax.experimental.pallas import tpu_sc as plsc

B = 1024
P = 100000
D = 128
K = 8
EPS = 1e-8
PB = 512
NB = (P + PB - 1) // PB
P_PAD = NB * PB
LASTV = P - (NB - 1) * PB
NPADS = float(P_PAD - P)
NEG = float("-inf")
BIGID = 1e9


def _scores_body(ppg2_ref, keys_ref, sc_ref, bm_ref, z_ref, s_ref,
                 ppgn_s, z_s, s_s, m_s):
    i = pl.program_id(0)

    @pl.when(i == 0)
    def _init():
        q = ppg2_ref[...]
        qn = jnp.maximum(jnp.sqrt(jnp.sum(q * q, axis=1, keepdims=True)), EPS)
        ppgn_s[...] = q / qn
        z_s[...] = jnp.zeros((B, D), jnp.float32)
        s_s[...] = jnp.zeros((B, D), jnp.float32)

    kblk = keys_ref[...]
    kn = jnp.maximum(jnp.sqrt(jnp.sum(kblk * kblk, axis=1, keepdims=True)), EPS)
    kbn = kblk / kn
    cos = lax.dot_general(ppgn_s[...], kbn, (((1,), (1,)), ((), ())),
                          preferred_element_type=jnp.float32)
    sc_ref[...] = cos

    t = -cos
    e = jnp.exp(t)
    te = t * e
    z_s[...] += (e[:, 0:128] + e[:, 128:256]) + (e[:, 256:384] + e[:, 384:512])
    s_s[...] += (te[:, 0:128] + te[:, 128:256]) + (te[:, 256:384] + te[:, 384:512])

    bm = jnp.max(cos, axis=1, keepdims=True)
    li = lax.broadcasted_iota(jnp.int32, (B, NB), 1)
    m_s[...] = jnp.where(li == i, bm, m_s[...])

    @pl.when(i == NB - 1)
    def _fin():
        ci = lax.broadcasted_iota(jnp.int32, (B, PB), 1)
        lbm = jnp.max(jnp.where(ci < LASTV, cos, NEG), axis=1, keepdims=True)
        bm_ref[...] = jnp.where(li == NB - 1, lbm, m_s[...])
        z_ref[...] = jnp.sum(z_s[...], axis=1, keepdims=True) - NPADS
        s_ref[...] = jnp.sum(s_s[...], axis=1, keepdims=True)


def _stream_scores(ppg2, keys_pad, interpret=False):
    return pl.pallas_call(
        _scores_body,
        grid=(NB,),
        in_specs=[
            pl.BlockSpec((B, D), lambda i: (0, 0)),
            pl.BlockSpec((PB, D), lambda i: (i, 0)),
        ],
        out_specs=[
            pl.BlockSpec((B, PB), lambda i: (0, i)),
            pl.BlockSpec((B, NB), lambda i: (0, 0)),
            pl.BlockSpec((B, 1), lambda i: (0, 0)),
            pl.BlockSpec((B, 1), lambda i: (0, 0)),
        ],
        out_shape=[
            jax.ShapeDtypeStruct((B, P_PAD), jnp.float32),
            jax.ShapeDtypeStruct((B, NB), jnp.float32),
            jax.ShapeDtypeStruct((B, 1), jnp.float32),
            jax.ShapeDtypeStruct((B, 1), jnp.float32),
        ],
        scratch_shapes=[
            pltpu.VMEM((B, D), jnp.float32),
            pltpu.VMEM((B, D), jnp.float32),
            pltpu.VMEM((B, D), jnp.float32),
            pltpu.VMEM((B, NB), jnp.float32),
        ],
        interpret=interpret,
    )(ppg2, keys_pad)


def _blocks_body(bm_ref, bid_ref, row_ref):
    m = bm_ref[...]
    gid = lax.broadcasted_iota(jnp.int32, (B, NB), 1).astype(jnp.float32)
    ids = []
    for _ in range(K):
        mx = jnp.max(m, axis=1, keepdims=True)
        am = jnp.min(jnp.where(m == mx, gid, BIGID), axis=1, keepdims=True)
        m = jnp.where(gid == am, NEG, m)
        ids.append(am)
    bids = jnp.concatenate(ids, axis=1)
    bid_ref[...] = bids
    rid = lax.broadcasted_iota(jnp.int32, (B, K), 0)
    row_ref[...] = rid * NB + bids.astype(jnp.int32)


def _top_blocks(bm, interpret=False):
    return pl.pallas_call(
        _blocks_body,
        out_shape=[
            jax.ShapeDtypeStruct((B, K), jnp.float32),
            jax.ShapeDtypeStruct((B, K), jnp.int32),
        ],
        interpret=interpret,
    )(bm)


def _make_sc_gather(n_rows, width, chunk):
    NC, NS = 2, 16
    NW = NC * NS
    bpw = n_rows // NW
    nch = bpw // chunk
    mesh = plsc.VectorSubcoreMesh(core_axis_name="c", subcore_axis_name="s")

    @functools.partial(
        pl.kernel, mesh=mesh,
        out_type=jax.ShapeDtypeStruct((n_rows, width), jnp.float32),
        scratch_types=[
            pltpu.VMEM((chunk,), jnp.int32),
            pltpu.VMEM((chunk, width), jnp.float32),
            pltpu.SemaphoreType.DMA,
        ],
    )
    def gather_k(table_hbm, idx_hbm, out_hbm, idx_v, rows_v, sem):
        wid = lax.axis_index("s") * NC + lax.axis_index("c")
        for j in range(nch):
            base = wid * bpw + j * chunk
            pltpu.sync_copy(idx_hbm.at[pl.ds(base, chunk)], idx_v)
            pltpu.async_copy(table_hbm.at[idx_v], rows_v, sem).wait()
            pltpu.sync_copy(rows_v, out_hbm.at[pl.ds(base, chunk)])

    return gather_k


def _cand_body(cand_ref, bid_ref, bv_ref, bi_ref):
    bids = bid_ref[...]
    ci = lax.broadcasted_iota(jnp.int32, (B, PB), 1).astype(jnp.float32)
    gids = jnp.concatenate(
        [bids[:, k:k + 1] * float(PB) + ci for k in range(K)], axis=1)
    cand = cand_ref[...]
    cm = jnp.where(gids < float(P), cand, NEG)
    vals, ids = [], []
    for _ in range(K):
        mx = jnp.max(cm, axis=1, keepdims=True)
        am = jnp.min(jnp.where(cm == mx, gids, BIGID), axis=1, keepdims=True)
        cm = jnp.where(gids == am, NEG, cm)
        vals.append(mx)
        ids.append(am)
    bv_ref[...] = jnp.concatenate(vals, axis=1)
    bi_ref[...] = jnp.concatenate(ids, axis=1)


def _cand_topk(cand, bids, interpret=False):
    return pl.pallas_call(
        _cand_body,
        out_shape=[
            jax.ShapeDtypeStruct((B, K), jnp.float32),
            jax.ShapeDtypeStruct((B, K), jnp.float32),
        ],
        interpret=interpret,
    )(cand, bids)


def _combine_body(g_ref, ppg2_ref, w_ref, b_ref, bv_ref, z_ref, s_ref,
                  out_ref, sm_ref, ent_ref, acc_s):
    k = pl.program_id(0)
    w0 = w_ref[0]
    w1 = w_ref[1]
    w2 = w_ref[2]
    bb = b_ref[0]
    x = g_ref[...]
    zcol = jnp.zeros((B, 1), jnp.float32)
    xl = jnp.concatenate([zcol, x[:, :D - 1]], axis=1)
    xr = jnp.concatenate([x[:, 1:], zcol], axis=1)
    y = jnp.maximum(w0 * xl + w1 * x + w2 * xr + bb, 0.0)

    @pl.when(k == 0)
    def _init():
        acc_s[...] = y

    @pl.when(k > 0)
    def _acc():
        acc_s[...] += y

    @pl.when(k == K - 1)
    def _fin():
        out_ref[...] = ppg2_ref[...] + acc_s[...] * jnp.float32(1.0 / K)
        sm_ref[...] = jnp.mean(1.0 - bv_ref[...]).reshape(1, 1)
        z = z_ref[...]
        s = s_ref[...]
        ent = jnp.log(z) - s / z
        ent_ref[...] = jnp.mean(ent).reshape(1, 1)


def _combine(gathered, ppg2, conv_w, conv_b, bv, z, s, interpret=False):
    return pl.pallas_call(
        _combine_body,
        grid=(K,),
        in_specs=[
            pl.BlockSpec((B, D), lambda k: (k, 0)),
            pl.BlockSpec((B, D), lambda k: (0, 0)),
            pl.BlockSpec(memory_space=pltpu.SMEM),
            pl.BlockSpec(memory_space=pltpu.SMEM),
            pl.BlockSpec((B, K), lambda k: (0, 0)),
            pl.BlockSpec((B, 1), lambda k: (0, 0)),
            pl.BlockSpec((B, 1), lambda k: (0, 0)),
        ],
        out_specs=[
            pl.BlockSpec((B, D), lambda k: (0, 0)),
            pl.BlockSpec((1, 1), lambda k: (0, 0)),
            pl.BlockSpec((1, 1), lambda k: (0, 0)),
        ],
        out_shape=[
            jax.ShapeDtypeStruct((B, D), jnp.float32),
            jax.ShapeDtypeStruct((1, 1), jnp.float32),
            jax.ShapeDtypeStruct((1, 1), jnp.float32),
        ],
        scratch_shapes=[pltpu.VMEM((B, D), jnp.float32)],
        interpret=interpret,
    )(gathered, ppg2, conv_w, conv_b, bv, z, s)


def kernel(ppg, keys, prompt, conv_w, conv_b):
    ppg2 = ppg[:, 0, :]
    keys_pad = jnp.pad(keys, ((0, P_PAD - P), (0, 0)))

    scores, bm, z, s = _stream_scores(ppg2, keys_pad)

    bids, rows = _top_blocks(bm)
    rows_flat = rows.reshape(B * K)
    score_table = scores.reshape(B * NB, PB)
    cand = _make_sc_gather(B * K, PB, 128)(score_table, rows_flat)
    cand = cand.reshape(B, K * PB)

    bv, bi = _cand_topk(cand, bids)

    idx = bi.astype(jnp.int32)
    idx_flat = idx.T.reshape(B * K)
    gathered = _make_sc_gather(B * K, D, 256)(prompt, idx_flat)

    out2, sm, ent = _combine(gathered, ppg2, conv_w, conv_b, bv, z, s)
    return (out2[:, None, :], sm[0, 0], ent[0, 0])
